# baseline (speedup 1.0000x reference)
import functools

import jax
import jax.numpy as jnp
from jax import lax
from jax.experimental import pallas as pl
from jax.experimental.pallas import tpu as pltpu
from jax.experimental.pallas import tpu_sc as plsc

_NC = 2
_NS = 16
_NW = _NC * _NS
_LANES = 16
_K = 64


def _mesh():
    return plsc.VectorSubcoreMesh(
        core_axis_name="c", subcore_axis_name="s",
        num_cores=_NC, num_subcores=_NS)


@functools.lru_cache(maxsize=None)
def _make_seg_sum(n_edges, n_dst):
    per_tile = n_edges // _NS
    n_chunks = per_tile // _K
    rows_per_tile = n_dst // _NS
    n_zsteps = rows_per_tile // _LANES

    def body(table_hbm, src_hbm, dst_hbm, zrow_hbm,
             agg_hbm,
             acc_sh, srcs_v, dsts_v, idx_a, idx_b, dst_a, dst_b,
             io_idx_v, rows_a, rows_b, io_rows_v, sem_a, sem_b,
             ssem_a, ssem_b):
        c = lax.axis_index("c")
        s = lax.axis_index("s")
        r0 = s * rows_per_tile
        lanes = lax.iota(jnp.int32, _LANES)

        def fill_io_idx(first_row):
            io_idx_v[...] = lanes + first_row

        base = s * per_tile
        pltpu.sync_copy(src_hbm.at[pl.ds(base, per_tile)], srcs_v)
        pltpu.sync_copy(dst_hbm.at[pl.ds(base, per_tile)], dsts_v)
        pltpu.sync_copy(zrow_hbm, io_rows_v)

        def izero(z, carry):
            fill_io_idx(r0 + z * _LANES)
            pltpu.sync_copy(io_rows_v, acc_sh.at[io_idx_v])
            return carry

        lax.fori_loop(0, n_zsteps, izero, 0)
        plsc.subcore_barrier()

        def prep(goff, idx_ref, dst_ref):
            for i in range(_K // _LANES):
                sl = pl.ds(i * _LANES, _LANES)
                ssl = pl.ds(goff + i * _LANES, _LANES)
                idx_ref[sl] = srcs_v[ssl] * 2 + c
                dst_ref[sl] = dsts_v[ssl]

        def gather(idx_ref, rows_ref, sem):
            pltpu.async_copy(table_hbm.at[idx_ref], rows_ref, sem)

        def gwait(idx_ref, rows_ref, sem):
            pltpu.make_async_copy(table_hbm.at[idx_ref], rows_ref, sem).wait()

        def scat(rows_ref, dst_ref, ssem):
            pltpu.async_copy(rows_ref, acc_sh.at[dst_ref], ssem, add=True)

        def swait(rows_ref, dst_ref, ssem):
            pltpu.make_async_copy(rows_ref, acc_sh.at[dst_ref], ssem).wait()

        prep(0, idx_a, dst_a)
        gather(idx_a, rows_a, sem_a)
        prep(_K, idx_b, dst_b)
        gather(idx_b, rows_b, sem_b)

        def pair(k, carry):
            g = k * 2
            gwait(idx_a, rows_a, sem_a)
            scat(rows_a, dst_a, ssem_a)
            gwait(idx_b, rows_b, sem_b)
            scat(rows_b, dst_b, ssem_b)
            swait(rows_a, dst_a, ssem_a)
            prep((g + 2) * _K, idx_a, dst_a)
            gather(idx_a, rows_a, sem_a)
            swait(rows_b, dst_b, ssem_b)
            prep((g + 3) * _K, idx_b, dst_b)
            gather(idx_b, rows_b, sem_b)
            return carry

        lax.fori_loop(0, n_chunks // 2 - 1, pair, 0)
        gwait(idx_a, rows_a, sem_a)
        scat(rows_a, dst_a, ssem_a)
        gwait(idx_b, rows_b, sem_b)
        scat(rows_b, dst_b, ssem_b)
        swait(rows_a, dst_a, ssem_a)
        swait(rows_b, dst_b, ssem_b)
        plsc.subcore_barrier()

        def owrite(z, carry):
            fill_io_idx(r0 + z * _LANES)
            pltpu.async_copy(acc_sh.at[io_idx_v], io_rows_v, sem_a).wait()
            pltpu.sync_copy(io_rows_v,
                            agg_hbm.at[pl.ds(c * n_dst + r0 + z * _LANES,
                                             _LANES)])
            return carry

        lax.fori_loop(0, n_zsteps, owrite, 0)

    return pl.kernel(
        body,
        out_type=jax.ShapeDtypeStruct((2 * n_dst, 128), jnp.float32),
        mesh=_mesh(),
        scratch_types=(
            pltpu.VMEM_SHARED((n_dst, 128), jnp.float32),
            pltpu.VMEM((per_tile,), jnp.int32),
            pltpu.VMEM((per_tile,), jnp.int32),
            pltpu.VMEM((_K,), jnp.int32),
            pltpu.VMEM((_K,), jnp.int32),
            pltpu.VMEM((_K,), jnp.int32),
            pltpu.VMEM((_K,), jnp.int32),
            pltpu.VMEM((_LANES,), jnp.int32),
            pltpu.VMEM((_K, 128), jnp.float32),
            pltpu.VMEM((_K, 128), jnp.float32),
            pltpu.VMEM((_LANES, 128), jnp.float32),
            pltpu.SemaphoreType.DMA,
            pltpu.SemaphoreType.DMA,
            pltpu.SemaphoreType.DMA,
            pltpu.SemaphoreType.DMA,
        ),
        name=f"sc_seg_sum_{n_edges}_{n_dst}",
    )


@functools.lru_cache(maxsize=None)
def _make_seg_cnt(n_edges, n_dst):
    per_w = n_edges // _NW
    kc = 2 * _K
    n_chunks = per_w // kc
    rows_per_tile = n_dst // _NS
    n_zsteps = rows_per_tile // _LANES

    def body(dst_hbm, zrow_hbm, ones_hbm,
             cnt_hbm,
             cnt_sh, dsts_v, dst_a, dst_b, io_idx_v, ones_v, io_rows_v,
             sem_a, sem_b):
        c = lax.axis_index("c")
        s = lax.axis_index("s")
        r0 = s * rows_per_tile
        lanes = lax.iota(jnp.int32, _LANES)

        def fill_io_idx(first_row):
            io_idx_v[...] = lanes + first_row

        base = (c * _NS + s) * per_w
        pltpu.sync_copy(dst_hbm.at[pl.ds(base, per_w)], dsts_v)
        pltpu.sync_copy(zrow_hbm, io_rows_v)
        pltpu.sync_copy(ones_hbm, ones_v)

        def izero(z, carry):
            fill_io_idx(r0 + z * _LANES)
            pltpu.sync_copy(io_rows_v, cnt_sh.at[io_idx_v])
            return carry

        lax.fori_loop(0, n_zsteps, izero, 0)
        plsc.subcore_barrier()

        def prep(goff, dst_ref):
            for i in range(kc // _LANES):
                dst_ref[pl.ds(i * _LANES, _LANES)] = (
                    dsts_v[pl.ds(goff + i * _LANES, _LANES)])

        def cadd(dst_ref, sem):
            pltpu.async_copy(ones_v, cnt_sh.at[dst_ref], sem, add=True)

        def cwait(dst_ref, sem):
            pltpu.make_async_copy(ones_v, cnt_sh.at[dst_ref], sem).wait()

        prep(0, dst_a)
        cadd(dst_a, sem_a)

        def pair(k, carry):
            g = k * 2
            prep((g + 1) * kc, dst_b)
            cadd(dst_b, sem_b)
            cwait(dst_a, sem_a)
            prep((g + 2) * kc, dst_a)
            cadd(dst_a, sem_a)
            cwait(dst_b, sem_b)
            return carry

        lax.fori_loop(0, n_chunks // 2 - 1, pair, 0)
        prep((n_chunks - 1) * kc, dst_b)
        cadd(dst_b, sem_b)
        cwait(dst_a, sem_a)
        cwait(dst_b, sem_b)
        plsc.subcore_barrier()

        def owrite(z, carry):
            fill_io_idx(r0 + z * _LANES)
            pltpu.async_copy(cnt_sh.at[io_idx_v], io_rows_v, sem_a).wait()
            pltpu.sync_copy(io_rows_v,
                            cnt_hbm.at[pl.ds(c * n_dst + r0 + z * _LANES,
                                             _LANES)])
            return carry

        lax.fori_loop(0, n_zsteps, owrite, 0)

    return pl.kernel(
        body,
        out_type=jax.ShapeDtypeStruct((2 * n_dst, 128), jnp.float32),
        mesh=_mesh(),
        scratch_types=(
            pltpu.VMEM_SHARED((n_dst, 128), jnp.float32),
            pltpu.VMEM((per_w,), jnp.int32),
            pltpu.VMEM((kc,), jnp.int32),
            pltpu.VMEM((kc,), jnp.int32),
            pltpu.VMEM((_LANES,), jnp.int32),
            pltpu.VMEM((kc, 128), jnp.float32),
            pltpu.VMEM((_LANES, 128), jnp.float32),
            pltpu.SemaphoreType.DMA,
            pltpu.SemaphoreType.DMA,
        ),
        name=f"sc_seg_cnt_{n_edges}_{n_dst}",
    )


def _tc_body(alo_ref, ahi_ref, cl_ref, ch_ref, x_ref, wl_ref, b_ref,
             wr_ref, out_ref, *, act):
    cnt = cl_ref[:, 0:1] + ch_ref[:, 0:1]
    inv = 1.0 / jnp.maximum(cnt, 1.0)
    agg = jnp.concatenate([alo_ref[...], ahi_ref[...]], axis=1)
    z = jnp.dot(agg, wl_ref[...], preferred_element_type=jnp.float32) * inv
    z = z + jnp.dot(x_ref[...], wr_ref[...], preferred_element_type=jnp.float32)
    z = z + b_ref[...]
    if act == "relu":
        out_ref[...] = jnp.maximum(z, 0.0)
    else:
        m = jnp.max(z, axis=-1, keepdims=True)
        e = jnp.exp(z - m)
        lse = jnp.log(jnp.sum(e, axis=-1, keepdims=True)) + m
        out_ref[...] = z - lse


@functools.lru_cache(maxsize=None)
def _make_tc_layer(n_dst, act):
    bm = 256
    nblk = n_dst // bm
    return pl.pallas_call(
        functools.partial(_tc_body, act=act),
        grid=(nblk,),
        in_specs=[
            pl.BlockSpec((bm, 128), lambda i: (i, 0)),
            pl.BlockSpec((bm, 128), lambda i, n=nblk: (i + n, 0)),
            pl.BlockSpec((bm, 128), lambda i: (i, 0)),
            pl.BlockSpec((bm, 128), lambda i, n=nblk: (i + n, 0)),
            pl.BlockSpec((bm, 256), lambda i: (i, 0)),
            pl.BlockSpec((256, 256), lambda i: (0, 0)),
            pl.BlockSpec((1, 256), lambda i: (0, 0)),
            pl.BlockSpec((256, 256), lambda i: (0, 0)),
        ],
        out_specs=pl.BlockSpec((bm, 256), lambda i: (i, 0)),
        out_shape=jax.ShapeDtypeStruct((n_dst, 256), jnp.float32),
    )


_SIZE1 = 10240
_SIZE2 = 1024
_E1 = 163840
_E2 = 16384


def kernel(x, edge_index1, edge_index2, size1_dst, size2_dst,
           W_l1, b_l1, W_r1, W_l2, b_l2, W_r2):
    zrow = jnp.zeros((_LANES, 128), jnp.float32)
    ones = jnp.ones((2 * _K, 128), jnp.float32)

    agg1 = _make_seg_sum(_E1, _SIZE1)(
        x.reshape(-1, 128), edge_index1[0], edge_index1[1], zrow)
    cnt1 = _make_seg_cnt(_E1, _SIZE1)(edge_index1[1], zrow, ones)
    cnt2 = _make_seg_cnt(_E2, _SIZE2)(edge_index2[1], zrow, ones)
    h = _make_tc_layer(_SIZE1, "relu")(
        agg1, agg1, cnt1, cnt1, x, W_l1, b_l1.reshape(1, -1), W_r1)

    agg2 = _make_seg_sum(_E2, _SIZE2)(
        h.reshape(-1, 128), edge_index2[0], edge_index2[1], zrow)
    out = _make_tc_layer(_SIZE2, "logsm")(
        agg2, agg2, cnt2, cnt2, h, W_l2, b_l2.reshape(1, -1), W_r2)
    return out


# speedup vs baseline: 1.1604x; 1.1604x over previous
import functools

import jax
import jax.numpy as jnp
from jax import lax
from jax.experimental import pallas as pl
from jax.experimental.pallas import tpu as pltpu
from jax.experimental.pallas import tpu_sc as plsc

_NC = 2
_NS = 16
_NW = _NC * _NS
_LANES = 16
_K = 64


def _mesh():
    return plsc.VectorSubcoreMesh(
        core_axis_name="c", subcore_axis_name="s",
        num_cores=_NC, num_subcores=_NS)


@functools.lru_cache(maxsize=None)
def _make_seg_sum(n_edges, n_dst):
    per_tile = n_edges // _NS
    n_chunks = per_tile // _K
    rows_per_tile = n_dst // _NS
    zch = min(64, rows_per_tile)
    n_zsteps = rows_per_tile // zch

    def body(table_hbm, src_hbm, dst_hbm, zrow_hbm,
             agg_hbm,
             acc_sh, srcs_v, dsts_v, idx_a, idx_b, dst_a, dst_b,
             io_idx_v, rows_a, rows_b, io_rows_v, sem_a, sem_b):
        c = lax.axis_index("c")
        s = lax.axis_index("s")
        r0 = s * rows_per_tile
        lanes = lax.iota(jnp.int32, _LANES)

        def fill_io_idx(first_row):
            for i in range(zch // _LANES):
                io_idx_v[pl.ds(i * _LANES, _LANES)] = (
                    lanes + (first_row + i * _LANES))

        base = s * per_tile
        pltpu.sync_copy(src_hbm.at[pl.ds(base, per_tile)], srcs_v)
        pltpu.sync_copy(dst_hbm.at[pl.ds(base, per_tile)], dsts_v)
        pltpu.sync_copy(zrow_hbm, io_rows_v)

        def izero(z, carry):
            fill_io_idx(r0 + z * zch)
            pltpu.sync_copy(io_rows_v, acc_sh.at[io_idx_v])
            return carry

        lax.fori_loop(0, n_zsteps, izero, 0)
        plsc.subcore_barrier()

        def prep(goff, idx_ref, dst_ref):
            for i in range(_K // _LANES):
                sl = pl.ds(i * _LANES, _LANES)
                ssl = pl.ds(goff + i * _LANES, _LANES)
                idx_ref[sl] = srcs_v[ssl] * 2 + c
                dst_ref[sl] = dsts_v[ssl]

        def gather(idx_ref, rows_ref, sem):
            pltpu.async_copy(table_hbm.at[idx_ref], rows_ref, sem)

        def gwait(idx_ref, rows_ref, sem):
            pltpu.make_async_copy(table_hbm.at[idx_ref], rows_ref, sem).wait()

        prep(0, idx_a, dst_a)
        gather(idx_a, rows_a, sem_a)

        def pair(k, carry):
            g = k * 2
            prep((g + 1) * _K, idx_b, dst_b)
            gather(idx_b, rows_b, sem_b)
            gwait(idx_a, rows_a, sem_a)
            pltpu.sync_copy(rows_a, acc_sh.at[dst_a], add=True)
            prep((g + 2) * _K, idx_a, dst_a)
            gather(idx_a, rows_a, sem_a)
            gwait(idx_b, rows_b, sem_b)
            pltpu.sync_copy(rows_b, acc_sh.at[dst_b], add=True)
            return carry

        lax.fori_loop(0, n_chunks // 2 - 1, pair, 0)
        prep((n_chunks - 1) * _K, idx_b, dst_b)
        gather(idx_b, rows_b, sem_b)
        gwait(idx_a, rows_a, sem_a)
        pltpu.sync_copy(rows_a, acc_sh.at[dst_a], add=True)
        gwait(idx_b, rows_b, sem_b)
        pltpu.sync_copy(rows_b, acc_sh.at[dst_b], add=True)
        plsc.subcore_barrier()

        def owrite(z, carry):
            fill_io_idx(r0 + z * zch)
            pltpu.async_copy(acc_sh.at[io_idx_v], io_rows_v, sem_a).wait()
            pltpu.sync_copy(io_rows_v,
                            agg_hbm.at[pl.ds(c * n_dst + r0 + z * zch,
                                             zch)])
            return carry

        lax.fori_loop(0, n_zsteps, owrite, 0)

    return pl.kernel(
        body,
        out_type=jax.ShapeDtypeStruct((2 * n_dst, 128), jnp.float32),
        mesh=_mesh(),
        scratch_types=(
            pltpu.VMEM_SHARED((n_dst, 128), jnp.float32),
            pltpu.VMEM((per_tile,), jnp.int32),
            pltpu.VMEM((per_tile,), jnp.int32),
            pltpu.VMEM((_K,), jnp.int32),
            pltpu.VMEM((_K,), jnp.int32),
            pltpu.VMEM((_K,), jnp.int32),
            pltpu.VMEM((_K,), jnp.int32),
            pltpu.VMEM((zch,), jnp.int32),
            pltpu.VMEM((_K, 128), jnp.float32),
            pltpu.VMEM((_K, 128), jnp.float32),
            pltpu.VMEM((zch, 128), jnp.float32),
            pltpu.SemaphoreType.DMA,
            pltpu.SemaphoreType.DMA,
        ),
        name=f"sc_seg_sum_{n_edges}_{n_dst}",
    )


@functools.lru_cache(maxsize=None)
def _make_seg_cnt(n_edges, n_dst):
    per_w = n_edges // _NW
    n_chunks = per_w // _K
    rows_per_tile = n_dst // _NS
    zch = min(64, rows_per_tile)
    n_zsteps = rows_per_tile // zch

    def body(dst_hbm, zrow_hbm, ones_hbm,
             cnt_hbm,
             cnt_sh, dsts_v, dst_a, dst_b, io_idx_v, ones_v, io_rows_v,
             sem_a, sem_b):
        c = lax.axis_index("c")
        s = lax.axis_index("s")
        r0 = s * rows_per_tile
        lanes = lax.iota(jnp.int32, _LANES)

        def fill_io_idx(first_row):
            for i in range(zch // _LANES):
                io_idx_v[pl.ds(i * _LANES, _LANES)] = (
                    lanes + (first_row + i * _LANES))

        base = (c * _NS + s) * per_w
        pltpu.sync_copy(dst_hbm.at[pl.ds(base, per_w)], dsts_v)
        pltpu.sync_copy(zrow_hbm, io_rows_v)
        pltpu.sync_copy(ones_hbm, ones_v)

        def izero(z, carry):
            fill_io_idx(r0 + z * zch)
            pltpu.sync_copy(io_rows_v, cnt_sh.at[io_idx_v])
            return carry

        lax.fori_loop(0, n_zsteps, izero, 0)
        plsc.subcore_barrier()

        def prep(goff, dst_ref):
            for i in range(_K // _LANES):
                dst_ref[pl.ds(i * _LANES, _LANES)] = (
                    dsts_v[pl.ds(goff + i * _LANES, _LANES)])

        def cadd(dst_ref, sem):
            pltpu.async_copy(ones_v, cnt_sh.at[dst_ref], sem, add=True)

        def cwait(dst_ref, sem):
            pltpu.make_async_copy(ones_v, cnt_sh.at[dst_ref], sem).wait()

        prep(0, dst_a)
        cadd(dst_a, sem_a)

        def pair(k, carry):
            g = k * 2
            prep((g + 1) * _K, dst_b)
            cadd(dst_b, sem_b)
            cwait(dst_a, sem_a)
            prep((g + 2) * _K, dst_a)
            cadd(dst_a, sem_a)
            cwait(dst_b, sem_b)
            return carry

        lax.fori_loop(0, n_chunks // 2 - 1, pair, 0)
        prep((n_chunks - 1) * _K, dst_b)
        cadd(dst_b, sem_b)
        cwait(dst_a, sem_a)
        cwait(dst_b, sem_b)
        plsc.subcore_barrier()

        def owrite(z, carry):
            fill_io_idx(r0 + z * zch)
            pltpu.async_copy(cnt_sh.at[io_idx_v], io_rows_v, sem_a).wait()
            pltpu.sync_copy(io_rows_v,
                            cnt_hbm.at[pl.ds(c * n_dst + r0 + z * zch,
                                             zch)])
            return carry

        lax.fori_loop(0, n_zsteps, owrite, 0)

    return pl.kernel(
        body,
        out_type=jax.ShapeDtypeStruct((2 * n_dst, 128), jnp.float32),
        mesh=_mesh(),
        scratch_types=(
            pltpu.VMEM_SHARED((n_dst, 128), jnp.float32),
            pltpu.VMEM((per_w,), jnp.int32),
            pltpu.VMEM((_K,), jnp.int32),
            pltpu.VMEM((_K,), jnp.int32),
            pltpu.VMEM((zch,), jnp.int32),
            pltpu.VMEM((_K, 128), jnp.float32),
            pltpu.VMEM((zch, 128), jnp.float32),
            pltpu.SemaphoreType.DMA,
            pltpu.SemaphoreType.DMA,
        ),
        name=f"sc_seg_cnt_{n_edges}_{n_dst}",
    )


def _tc_body(alo_ref, ahi_ref, cl_ref, ch_ref, x_ref, wl_ref, b_ref,
             wr_ref, out_ref, *, act):
    cnt = cl_ref[:, 0:1] + ch_ref[:, 0:1]
    inv = 1.0 / jnp.maximum(cnt, 1.0)
    agg = jnp.concatenate([alo_ref[...], ahi_ref[...]], axis=1)
    z = jnp.dot(agg, wl_ref[...], preferred_element_type=jnp.float32) * inv
    z = z + jnp.dot(x_ref[...], wr_ref[...], preferred_element_type=jnp.float32)
    z = z + b_ref[...]
    if act == "relu":
        out_ref[...] = jnp.maximum(z, 0.0)
    else:
        m = jnp.max(z, axis=-1, keepdims=True)
        e = jnp.exp(z - m)
        lse = jnp.log(jnp.sum(e, axis=-1, keepdims=True)) + m
        out_ref[...] = z - lse


@functools.lru_cache(maxsize=None)
def _make_tc_layer(n_dst, act):
    bm = 256
    nblk = n_dst // bm
    return pl.pallas_call(
        functools.partial(_tc_body, act=act),
        grid=(nblk,),
        in_specs=[
            pl.BlockSpec((bm, 128), lambda i: (i, 0)),
            pl.BlockSpec((bm, 128), lambda i, n=nblk: (i + n, 0)),
            pl.BlockSpec((bm, 128), lambda i: (i, 0)),
            pl.BlockSpec((bm, 128), lambda i, n=nblk: (i + n, 0)),
            pl.BlockSpec((bm, 256), lambda i: (i, 0)),
            pl.BlockSpec((256, 256), lambda i: (0, 0)),
            pl.BlockSpec((1, 256), lambda i: (0, 0)),
            pl.BlockSpec((256, 256), lambda i: (0, 0)),
        ],
        out_specs=pl.BlockSpec((bm, 256), lambda i: (i, 0)),
        out_shape=jax.ShapeDtypeStruct((n_dst, 256), jnp.float32),
    )


_SIZE1 = 10240
_SIZE2 = 1024
_E1 = 163840
_E2 = 16384


def kernel(x, edge_index1, edge_index2, size1_dst, size2_dst,
           W_l1, b_l1, W_r1, W_l2, b_l2, W_r2):
    zrow = jnp.zeros((64, 128), jnp.float32)
    ones = jnp.ones((_K, 128), jnp.float32)

    agg1 = _make_seg_sum(_E1, _SIZE1)(
        x.reshape(-1, 128), edge_index1[0], edge_index1[1], zrow)
    cnt1 = _make_seg_cnt(_E1, _SIZE1)(edge_index1[1], zrow, ones)
    h = _make_tc_layer(_SIZE1, "relu")(
        agg1, agg1, cnt1, cnt1, x, W_l1, b_l1.reshape(1, -1), W_r1)

    agg2 = _make_seg_sum(_E2, _SIZE2)(
        h.reshape(-1, 128), edge_index2[0], edge_index2[1], zrow)
    cnt2 = _make_seg_cnt(_E2, _SIZE2)(edge_index2[1], zrow, ones)
    out = _make_tc_layer(_SIZE2, "logsm")(
        agg2, agg2, cnt2, cnt2, h, W_l2, b_l2.reshape(1, -1), W_r2)
    return out
